# Initial kernel scaffold; baseline (speedup 1.0000x reference)
import jax
import jax.numpy as jnp
from jax.experimental import pallas as pl


def kernel(input_ids, emb, W1, b1, W2, b2, W3, b3, W4, b4):
    raise NotImplementedError("write your pallas kernel here")


# speedup vs baseline: 11.9400x; 11.9400x over previous
import functools

import jax
import jax.numpy as jnp
from jax import lax
from jax.experimental import pallas as pl
from jax.experimental.pallas import tpu as pltpu
from jax.experimental.pallas import tpu_sc as plsc

B = 16384
S = 50
D = 128
NV = D // 16
NW = 32
RPW = B // NW
CB = 8
NCHUNK = RPW // CB
INV_S = 1.0 / S


def _pooled_sc(ids, emb):
    mesh = plsc.VectorSubcoreMesh(core_axis_name="core", subcore_axis_name="subcore")

    @functools.partial(
        pl.kernel,
        out_type=jax.ShapeDtypeStruct((B, D), jnp.float32),
        mesh=mesh,
        scratch_types=[
            pltpu.VMEM((2, CB, S), jnp.int32),
            pltpu.VMEM((2, CB, S, D), jnp.float32),
            pltpu.VMEM((2, CB, D), jnp.float32),
            pltpu.SemaphoreType.DMA,
            pltpu.SemaphoreType.DMA,
        ],
    )
    def kern(ids_hbm, emb_hbm, out_hbm, idx_v, rows_v, out_v, sem0, sem1):
        wid = lax.axis_index("core") * 16 + lax.axis_index("subcore")
        base = wid * RPW
        sems = (sem0, sem1)

        def fire(buf, c, sem):
            row0 = base + c * CB
            pltpu.sync_copy(ids_hbm.at[pl.ds(row0, CB), :], idx_v.at[buf])
            return [
                pltpu.async_copy(emb_hbm.at[idx_v.at[buf, b]], rows_v.at[buf, b], sem)
                for b in range(CB)
            ]

        def accum_store(buf, c):
            row0 = base + c * CB
            for b in range(CB):
                def sbody(s, acc, _b=b):
                    return tuple(
                        acc[j] + rows_v[buf, _b, s, pl.ds(16 * j, 16)]
                        for j in range(NV)
                    )
                acc = lax.fori_loop(
                    0, S, sbody,
                    tuple(jnp.zeros((16,), jnp.float32) for _ in range(NV)),
                )
                for j in range(NV):
                    out_v[buf, b, pl.ds(16 * j, 16)] = acc[j] * INV_S
            pltpu.sync_copy(out_v.at[buf], out_hbm.at[pl.ds(row0, CB), :])

        for d in fire(0, 0, sems[0]):
            d.wait()

        @pl.loop(0, NCHUNK, step=2)
        def _(c):
            d1 = fire(1, c + 1, sems[1])
            accum_store(0, c)
            for d in d1:
                d.wait()

            @pl.when(c + 2 < NCHUNK)
            def _():
                d0 = fire(0, c + 2, sems[0])
                accum_store(1, c + 1)
                for d in d0:
                    d.wait()

            @pl.when(c + 2 >= NCHUNK)
            def _():
                accum_store(1, c + 1)

    return kern(ids, emb)


def _mlp_body(x_ref, w1, b1r, w2, b2r, w3, b3r, w4, b4r, o_ref):
    hi = jax.lax.Precision.HIGHEST
    x = x_ref[...]
    h = jnp.maximum(
        jnp.dot(x, w1[...], precision=hi, preferred_element_type=jnp.float32)
        + b1r[...], 0.0)
    h = jnp.maximum(
        jnp.dot(h, w2[...], precision=hi, preferred_element_type=jnp.float32)
        + b2r[...], 0.0)
    h = jnp.maximum(
        jnp.dot(h, w3[...], precision=hi, preferred_element_type=jnp.float32)
        + b3r[...], 0.0)
    o_ref[...] = (
        jnp.dot(h, w4[...], precision=hi, preferred_element_type=jnp.float32)
        + b4r[...])


def _mlp_tc(x, W1, b1, W2, b2, W3, b3, W4, b4):
    BM = 2048
    full = lambda shape: pl.BlockSpec(shape, lambda i: (0, 0))
    return pl.pallas_call(
        _mlp_body,
        grid=(B // BM,),
        in_specs=[
            pl.BlockSpec((BM, D), lambda i: (i, 0)),
            full((D, 128)), full((1, 128)),
            full((128, 128)), full((1, 128)),
            full((128, 32)), full((1, 32)),
            full((32, 2)), full((1, 2)),
        ],
        out_specs=pl.BlockSpec((BM, 2), lambda i: (i, 0)),
        out_shape=jax.ShapeDtypeStruct((B, 2), jnp.float32),
    )(x, W1, b1.reshape(1, -1), W2, b2.reshape(1, -1),
      W3, b3.reshape(1, -1), W4, b4.reshape(1, -1))


def kernel(input_ids, emb, W1, b1, W2, b2, W3, b3, W4, b4):
    ids = input_ids.astype(jnp.int32)
    pooled = _pooled_sc(ids, emb)
    return _mlp_tc(pooled, W1, b1, W2, b2, W3, b3, W4, b4)


# speedup vs baseline: 13.5717x; 1.1367x over previous
import functools

import jax
import jax.numpy as jnp
from jax import lax
from jax.experimental import pallas as pl
from jax.experimental.pallas import tpu as pltpu
from jax.experimental.pallas import tpu_sc as plsc

B = 16384
S = 50
D = 128
NV = D // 16
NW = 32
RPW = B // NW
CB = 8
NCHUNK = RPW // CB
INV_S = 1.0 / S


def _pooled_sc(ids, emb):
    mesh = plsc.VectorSubcoreMesh(core_axis_name="core", subcore_axis_name="subcore")

    @functools.partial(
        pl.kernel,
        out_type=jax.ShapeDtypeStruct((B, D), jnp.float32),
        mesh=mesh,
        scratch_types=[
            pltpu.VMEM((2, CB, S), jnp.int32),
            pltpu.VMEM((2, CB, S, D), jnp.float32),
            pltpu.VMEM((2, CB, D), jnp.float32),
            pltpu.SemaphoreType.DMA,
            pltpu.SemaphoreType.DMA,
            pltpu.SemaphoreType.DMA,
            pltpu.SemaphoreType.DMA,
        ],
    )
    def kern(ids_hbm, emb_hbm, out_hbm, idx_v, rows_v, out_v, g0, g1, i0, i1):
        wid = lax.axis_index("core") * 16 + lax.axis_index("subcore")
        base = wid * RPW
        gsem = (g0, g1)
        isem = (i0, i1)

        def idx_start(buf, c):
            row0 = base + c * CB
            pltpu.async_copy(ids_hbm.at[pl.ds(row0, CB), :], idx_v.at[buf],
                             isem[buf])

        def idx_wait(buf, c):
            row0 = base + c * CB
            pltpu.make_async_copy(ids_hbm.at[pl.ds(row0, CB), :],
                                  idx_v.at[buf], isem[buf]).wait()

        def fire(buf):
            return [
                pltpu.async_copy(emb_hbm.at[idx_v.at[buf, b]], rows_v.at[buf, b],
                                 gsem[buf])
                for b in range(CB)
            ]

        def accum_store(buf, c):
            row0 = base + c * CB
            for b in range(CB):
                def sbody(s, acc, _b=b):
                    for u in range(5):
                        acc = tuple(
                            acc[j] + rows_v[buf, _b, s * 5 + u, pl.ds(16 * j, 16)]
                            for j in range(NV)
                        )
                    return acc
                acc = lax.fori_loop(
                    0, S // 5, sbody,
                    tuple(jnp.zeros((16,), jnp.float32) for _ in range(NV)),
                )
                for j in range(NV):
                    out_v[buf, b, pl.ds(16 * j, 16)] = acc[j] * INV_S
            pltpu.sync_copy(out_v.at[buf], out_hbm.at[pl.ds(row0, CB), :])

        def substep(buf, c, fire_next, start_idx2):
            nxt = 1 - buf
            d = []
            if fire_next:
                idx_wait(nxt, c + 1)
                d = fire(nxt)
            if start_idx2:
                idx_start(buf, c + 2)
            accum_store(buf, c)
            for dd in d:
                dd.wait()

        idx_start(0, 0)
        idx_wait(0, 0)
        d0 = fire(0)
        idx_start(1, 1)
        for dd in d0:
            dd.wait()

        @pl.loop(0, NCHUNK - 2, step=2)
        def _(c):
            substep(0, c, True, True)
            substep(1, c + 1, True, True)

        substep(0, NCHUNK - 2, True, False)
        substep(1, NCHUNK - 1, False, False)

    return kern(ids, emb)


def _mlp_body(x_ref, w1, b1r, w2, b2r, w3, b3r, w4, b4r, o_ref):
    hi = jax.lax.Precision.HIGHEST
    x = x_ref[...]
    h = jnp.maximum(
        jnp.dot(x, w1[...], precision=hi, preferred_element_type=jnp.float32)
        + b1r[...], 0.0)
    h = jnp.maximum(
        jnp.dot(h, w2[...], precision=hi, preferred_element_type=jnp.float32)
        + b2r[...], 0.0)
    h = jnp.maximum(
        jnp.dot(h, w3[...], precision=hi, preferred_element_type=jnp.float32)
        + b3r[...], 0.0)
    o_ref[...] = (
        jnp.dot(h, w4[...], precision=hi, preferred_element_type=jnp.float32)
        + b4r[...])


def _mlp_tc(x, W1, b1, W2, b2, W3, b3, W4, b4):
    BM = 2048
    full = lambda shape: pl.BlockSpec(shape, lambda i: (0, 0))
    return pl.pallas_call(
        _mlp_body,
        grid=(B // BM,),
        in_specs=[
            pl.BlockSpec((BM, D), lambda i: (i, 0)),
            full((D, 128)), full((1, 128)),
            full((128, 128)), full((1, 128)),
            full((128, 32)), full((1, 32)),
            full((32, 2)), full((1, 2)),
        ],
        out_specs=pl.BlockSpec((BM, 2), lambda i: (i, 0)),
        out_shape=jax.ShapeDtypeStruct((B, 2), jnp.float32),
    )(x, W1, b1.reshape(1, -1), W2, b2.reshape(1, -1),
      W3, b3.reshape(1, -1), W4, b4.reshape(1, -1))


def kernel(input_ids, emb, W1, b1, W2, b2, W3, b3, W4, b4):
    ids = input_ids.astype(jnp.int32)
    pooled = _pooled_sc(ids, emb)
    return _mlp_tc(pooled, W1, b1, W2, b2, W3, b3, W4, b4)
